# Initial kernel scaffold; baseline (speedup 1.0000x reference)
import jax
import jax.numpy as jnp
from jax import lax
from jax.experimental import pallas as pl
from jax.experimental.pallas import tpu as pltpu
from jax.experimental.pallas import tpu_sc as plsc

_NUM_CORES = 2
_NUM_SUBCORES = 16
_NUM_WORKERS = _NUM_CORES * _NUM_SUBCORES

_B = 16384 * 50
_D = 32
_BPW = _B // _NUM_WORKERS
_CHUNK = 1600
_NCHUNK = _BPW // _CHUNK


def _lookup_body(idx_hbm, table_hbm, out_hbm, idx_v, rows_v, sem):
    wid = lax.axis_index("s") * _NUM_CORES + lax.axis_index("c")
    base = wid * _BPW

    def body(j, carry):
        off = base + j * _CHUNK
        pltpu.sync_copy(idx_hbm.at[pl.ds(off, _CHUNK)], idx_v)
        pltpu.async_copy(table_hbm.at[idx_v], rows_v, sem).wait()
        pltpu.sync_copy(rows_v, out_hbm.at[pl.ds(off, _CHUNK)])
        return carry

    lax.fori_loop(0, _NCHUNK, body, 0)


_lookup = pl.kernel(
    _lookup_body,
    out_type=jax.ShapeDtypeStruct((_B, _D), jnp.float32),
    mesh=plsc.VectorSubcoreMesh(core_axis_name="c", subcore_axis_name="s"),
    scratch_types=[
        pltpu.VMEM((_CHUNK,), jnp.int32),
        pltpu.VMEM((_CHUNK, _D), jnp.float32),
        pltpu.SemaphoreType.DMA,
    ],
)


@jax.jit
def kernel(idx, embedding):
    flat = idx.reshape(-1).astype(jnp.int32)
    out = _lookup(flat, embedding)
    return out.reshape(idx.shape + (embedding.shape[1],))


# speedup vs baseline: 1.1030x; 1.1030x over previous
import jax
import jax.numpy as jnp
from jax import lax
from jax.experimental import pallas as pl
from jax.experimental.pallas import tpu as pltpu
from jax.experimental.pallas import tpu_sc as plsc

_NUM_CORES = 2
_NUM_SUBCORES = 16
_NUM_WORKERS = _NUM_CORES * _NUM_SUBCORES

_B = 16384 * 50
_D = 32
_BPW = _B // _NUM_WORKERS
_CHUNK = 1600
_NCHUNK = _BPW // _CHUNK


def _lookup_body(idx_hbm, table_hbm, out_hbm, idx_v, rows_v, sem):
    wid = lax.axis_index("s") * _NUM_CORES + lax.axis_index("c")
    base = wid * _BPW

    def body(j, carry):
        off = base + j * _CHUNK
        pltpu.sync_copy(idx_hbm.at[pl.ds(off, _CHUNK)], idx_v)
        pltpu.async_copy(table_hbm.at[idx_v], rows_v, sem).wait()
        pltpu.sync_copy(rows_v, out_hbm.at[pl.ds(off, _CHUNK)])
        return carry

    lax.fori_loop(0, _NCHUNK, body, 0)


_lookup = pl.kernel(
    _lookup_body,
    out_type=jax.ShapeDtypeStruct((_B, _D), jnp.float32),
    mesh=plsc.VectorSubcoreMesh(core_axis_name="c", subcore_axis_name="s"),
    scratch_types=[
        pltpu.VMEM((_CHUNK,), jnp.int32),
        pltpu.VMEM((_CHUNK, _D), jnp.float32),
        pltpu.SemaphoreType.DMA,
    ],
    compiler_params=pltpu.CompilerParams(use_tc_tiling_on_sc=False),
)


@jax.jit
def kernel(idx, embedding):
    flat = idx.reshape(-1).astype(jnp.int32)
    out = _lookup(flat, embedding)
    return out.reshape(idx.shape + (embedding.shape[1],))


# speedup vs baseline: 1.1133x; 1.0093x over previous
import jax
import jax.numpy as jnp
from jax import lax
from jax.experimental import pallas as pl
from jax.experimental.pallas import tpu as pltpu
from jax.experimental.pallas import tpu_sc as plsc

_NUM_CORES = 2
_NUM_SUBCORES = 16
_NUM_WORKERS = _NUM_CORES * _NUM_SUBCORES

_B = 16384 * 50
_D = 32
_BPW = _B // _NUM_WORKERS
_CHUNK = 1600
_NCHUNK = _BPW // _CHUNK


def _lookup_body(idx_hbm, table_hbm, out_hbm, idx_v, rows0, rows1,
                 gsem0, gsem1, wsem0, wsem1):
    wid = lax.axis_index("s") * _NUM_CORES + lax.axis_index("c")
    base = wid * _BPW

    pltpu.sync_copy(idx_hbm.at[pl.ds(base, _BPW)], idx_v)

    rows = [rows0, rows1]
    gsem = [gsem0, gsem1]
    wsem = [wsem0, wsem1]
    gh = [None, None]
    wh = [None, None]

    gh[0] = pltpu.async_copy(
        table_hbm.at[idx_v.at[pl.ds(0, _CHUNK)]], rows[0], gsem[0])
    for j in range(_NCHUNK):
        cur = j & 1
        nxt = cur ^ 1
        if j + 1 < _NCHUNK:
            if wh[nxt] is not None:
                wh[nxt].wait()
            gh[nxt] = pltpu.async_copy(
                table_hbm.at[idx_v.at[pl.ds((j + 1) * _CHUNK, _CHUNK)]],
                rows[nxt], gsem[nxt])
        gh[cur].wait()
        wh[cur] = pltpu.async_copy(
            rows[cur], out_hbm.at[pl.ds(base + j * _CHUNK, _CHUNK)], wsem[cur])
    wh[0].wait()
    wh[1].wait()


_lookup = pl.kernel(
    _lookup_body,
    out_type=jax.ShapeDtypeStruct((_B, _D), jnp.float32),
    mesh=plsc.VectorSubcoreMesh(core_axis_name="c", subcore_axis_name="s"),
    scratch_types=[
        pltpu.VMEM((_BPW,), jnp.int32),
        pltpu.VMEM((_CHUNK, _D), jnp.float32),
        pltpu.VMEM((_CHUNK, _D), jnp.float32),
        pltpu.SemaphoreType.DMA,
        pltpu.SemaphoreType.DMA,
        pltpu.SemaphoreType.DMA,
        pltpu.SemaphoreType.DMA,
    ],
    compiler_params=pltpu.CompilerParams(use_tc_tiling_on_sc=False),
)


@jax.jit
def kernel(idx, embedding):
    flat = idx.reshape(-1).astype(jnp.int32)
    out = _lookup(flat, embedding)
    return out.reshape(idx.shape + (embedding.shape[1],))


# speedup vs baseline: 1.3599x; 1.2215x over previous
import jax
import jax.numpy as jnp
from jax import lax
from jax.experimental import pallas as pl
from jax.experimental.pallas import tpu as pltpu
from jax.experimental.pallas import tpu_sc as plsc

_NUM_CORES = 2
_NUM_SUBCORES = 16
_NUM_WORKERS = _NUM_CORES * _NUM_SUBCORES

_BATCH = 16384
_SEQ = 50
_D = 32
_BB = 128
_NBB = _BATCH // _BB
_BBW = _NBB // _NUM_WORKERS
_BPW = _BB * _BBW
_L = 16


def _lookup_body(idx_hbm, table_hbm, out_hbm, idx_v, rowid_v, rows_v, trans_v,
                 gsem, wsem):
    wid = lax.axis_index("s") * _NUM_CORES + lax.axis_index("c")
    b_base = wid * _BPW

    pltpu.sync_copy(idx_hbm.at[pl.ds(b_base * _SEQ, _BPW * _SEQ)], idx_v)

    lane = lax.iota(jnp.int32, _L)

    def unit(u, carry):
        s = u // _BBW
        btl = u - s * _BBW

        for lg in range(_BB // _L):
            pos = (btl * _BB + lg * _L) * _SEQ + s + lane * _SEQ
            v = plsc.load_gather(idx_v, [pos])
            rowid_v[pl.ds(lg * _L, _L)] = v

        pltpu.async_copy(table_hbm.at[rowid_v], rows_v, gsem).wait()

        for ct in range(_D // 8):
            for si in range(8):
                c = ct * 8 + si
                for lg in range(_BB // _L):
                    vals = plsc.load_gather(rows_v, [lg * _L + lane, lane * 0 + c])
                    trans_v[si, pl.ds(lg * _L, _L)] = vals
            bt = wid * _BBW + btl
            pltpu.async_copy(trans_v, out_hbm.at[s, ct, bt], wsem).wait()
        return carry

    lax.fori_loop(0, _SEQ * _BBW, unit, 0)


_lookup = pl.kernel(
    _lookup_body,
    out_type=jax.ShapeDtypeStruct((_SEQ, _D // 8, _NBB, 8, _BB), jnp.float32),
    mesh=plsc.VectorSubcoreMesh(core_axis_name="c", subcore_axis_name="s"),
    scratch_types=[
        pltpu.VMEM((_BPW * _SEQ,), jnp.int32),
        pltpu.VMEM((_BB,), jnp.int32),
        pltpu.VMEM((_BB, _D), jnp.float32),
        pltpu.VMEM((8, _BB), jnp.float32),
        pltpu.SemaphoreType.DMA,
        pltpu.SemaphoreType.DMA,
    ],
    compiler_params=pltpu.CompilerParams(
        use_tc_tiling_on_sc=False, needs_layout_passes=False
    ),
)


@jax.jit
def kernel(idx, embedding):
    flat = idx.reshape(-1).astype(jnp.int32)
    out5 = _lookup(flat, embedding)
    out = out5.transpose(2, 4, 0, 1, 3).reshape(_BATCH, _SEQ, _D)
    return out


# speedup vs baseline: 1.5095x; 1.1101x over previous
import jax
import jax.numpy as jnp
from jax import lax
from jax.experimental import pallas as pl
from jax.experimental.pallas import tpu as pltpu
from jax.experimental.pallas import tpu_sc as plsc

_NUM_CORES = 2
_NUM_SUBCORES = 16
_NUM_WORKERS = _NUM_CORES * _NUM_SUBCORES

_BATCH = 16384
_SEQ = 50
_D = 32
_BB = 128
_NBB = _BATCH // _BB
_BBW = _NBB // _NUM_WORKERS
_BPW = _BB * _BBW
_L = 16
_NU = _SEQ * _BBW


def _lookup_body(idx_hbm, table_hbm, out_hbm, idx_v, rid0, rid1, rows0, rows1,
                 tr0, tr1, gsem0, gsem1, wsem0, wsem1):
    wid = lax.axis_index("s") * _NUM_CORES + lax.axis_index("c")
    b_base = wid * _BPW
    bt_base = wid * _BBW

    pltpu.sync_copy(idx_hbm.at[pl.ds(b_base * _SEQ, _BPW * _SEQ)], idx_v)

    lane = lax.iota(jnp.int32, _L)
    rid = [rid0, rid1]
    rows = [rows0, rows1]
    tr = [tr0, tr1]
    gsem = [gsem0, gsem1]
    wsem = [wsem0, wsem1]

    def build_rowid(u, x):
        s = u // _BBW
        btl = u - s * _BBW
        for lg in range(_BB // _L):
            pos = (btl * _BB + lg * _L) * _SEQ + s + lane * _SEQ
            rid[x][pl.ds(lg * _L, _L)] = plsc.load_gather(idx_v, [pos])

    def issue_gather(x):
        pltpu.async_copy(table_hbm.at[rid[x]], rows[x], gsem[x])

    def wait_gather(x):
        pltpu.make_async_copy(table_hbm.at[rid[x]], rows[x], gsem[x]).wait()

    def transpose_unit(x):
        for ct in range(_D // 8):
            for si in range(8):
                c = ct * 8 + si
                cvec = lane * 0 + c
                for lg in range(_BB // _L):
                    vals = plsc.load_gather(rows[x], [lg * _L + lane, cvec])
                    tr[x][ct, si, pl.ds(lg * _L, _L)] = vals

    def issue_writes(u, x):
        s = u // _BBW
        bt = bt_base + (u - s * _BBW)
        for ct in range(_D // 8):
            pltpu.async_copy(tr[x].at[ct], out_hbm.at[s, ct, bt], wsem[x])

    def drain_writes(u, x):
        s = u // _BBW
        bt = bt_base + (u - s * _BBW)
        for ct in range(_D // 8):
            pltpu.make_async_copy(tr[x].at[ct], out_hbm.at[s, ct, bt],
                                  wsem[x]).wait()

    for u in range(2):
        build_rowid(u, u)
        issue_gather(u)
    for u in range(2):
        wait_gather(u)
        transpose_unit(u)
        issue_writes(u, u)
        build_rowid(u + 2, u)
        issue_gather(u)

    def body(t, carry):
        for k in range(2):
            u = 2 + 2 * t + k
            x = k
            wait_gather(x)
            drain_writes(u - 2, x)
            transpose_unit(x)
            issue_writes(u, x)
            u2 = lax.min(u + 2, _NU - 1)
            build_rowid(u2, x)
            issue_gather(x)
        return carry

    lax.fori_loop(0, (_NU - 2) // 2, body, 0)

    for k in range(2):
        wait_gather(k)
        drain_writes(_NU - 2 + k, k)


_lookup = pl.kernel(
    _lookup_body,
    out_type=jax.ShapeDtypeStruct((_SEQ, _D // 8, _NBB, 8, _BB), jnp.float32),
    mesh=plsc.VectorSubcoreMesh(core_axis_name="c", subcore_axis_name="s"),
    scratch_types=[
        pltpu.VMEM((_BPW * _SEQ,), jnp.int32),
        pltpu.VMEM((_BB,), jnp.int32),
        pltpu.VMEM((_BB,), jnp.int32),
        pltpu.VMEM((_BB, _D), jnp.float32),
        pltpu.VMEM((_BB, _D), jnp.float32),
        pltpu.VMEM((_D // 8, 8, _BB), jnp.float32),
        pltpu.VMEM((_D // 8, 8, _BB), jnp.float32),
        pltpu.SemaphoreType.DMA,
        pltpu.SemaphoreType.DMA,
        pltpu.SemaphoreType.DMA,
        pltpu.SemaphoreType.DMA,
    ],
    compiler_params=pltpu.CompilerParams(
        use_tc_tiling_on_sc=False, needs_layout_passes=False
    ),
)


@jax.jit
def kernel(idx, embedding):
    flat = idx.reshape(-1).astype(jnp.int32)
    out5 = _lookup(flat, embedding)
    out = out5.transpose(2, 4, 0, 1, 3).reshape(_BATCH, _SEQ, _D)
    return out


# speedup vs baseline: 1.8868x; 1.2499x over previous
import jax
import jax.numpy as jnp
from jax import lax
from jax.experimental import pallas as pl
from jax.experimental.pallas import tpu as pltpu
from jax.experimental.pallas import tpu_sc as plsc

_NUM_CORES = 2
_NUM_SUBCORES = 16
_NUM_WORKERS = _NUM_CORES * _NUM_SUBCORES

_BATCH = 16384
_SEQ = 50
_D = 32
_BB = 128
_NBB = _BATCH // _BB
_BBW = _NBB // _NUM_WORKERS
_BPW = _BB * _BBW
_L = 16
_NU = _SEQ * _BBW


def _lookup_body(idx_hbm, table_hbm, out_hbm, idx_v, rid0, rid1, rows0, rows1,
                 rp0, rp1, tr0, tr1, gsem0, gsem1, wsem0, wsem1):
    wid = lax.axis_index("s") * _NUM_CORES + lax.axis_index("c")
    b_base = wid * _BPW
    bt_base = wid * _BBW

    pltpu.sync_copy(idx_hbm.at[pl.ds(b_base * _SEQ, _BPW * _SEQ)], idx_v)

    lane = lax.iota(jnp.int32, _L)
    rid = [rid0, rid1]
    rows = [rows0, rows1]
    rows33 = [rp0, rp1]
    tr = [tr0, tr1]
    gsem = [gsem0, gsem1]
    wsem = [wsem0, wsem1]

    def build_rowid(u, x):
        s = u // _BBW
        btl = u - s * _BBW
        for lg in range(_BB // _L):
            pos = (btl * _BB + lg * _L) * _SEQ + s + lane * _SEQ
            rid[x][pl.ds(lg * _L, _L)] = plsc.load_gather(idx_v, [pos])

    def issue_gather(x):
        pltpu.async_copy(table_hbm.at[rid[x]], rows[x], gsem[x])

    def wait_gather(x):
        pltpu.make_async_copy(table_hbm.at[rid[x]], rows[x], gsem[x]).wait()

    def transpose_unit(x):
        for j in range(_BB):
            for h in range(_D // _L):
                rows33[x][j, pl.ds(h * _L, _L)] = rows[x][j, pl.ds(h * _L, _L)]
        for ct in range(_D // 8):
            for si in range(8):
                c = ct * 8 + si
                cvec = lane * 0 + c
                for lg in range(_BB // _L):
                    vals = plsc.load_gather(rows33[x], [lg * _L + lane, cvec])
                    tr[x][ct, si, pl.ds(lg * _L, _L)] = vals

    def issue_writes(u, x):
        s = u // _BBW
        bt = bt_base + (u - s * _BBW)
        for ct in range(_D // 8):
            pltpu.async_copy(tr[x].at[ct], out_hbm.at[s, ct, bt], wsem[x])

    def drain_writes(u, x):
        s = u // _BBW
        bt = bt_base + (u - s * _BBW)
        for ct in range(_D // 8):
            pltpu.make_async_copy(tr[x].at[ct], out_hbm.at[s, ct, bt],
                                  wsem[x]).wait()

    for u in range(2):
        build_rowid(u, u)
        issue_gather(u)
    for u in range(2):
        wait_gather(u)
        transpose_unit(u)
        issue_writes(u, u)
        build_rowid(u + 2, u)
        issue_gather(u)

    def body(t, carry):
        for k in range(2):
            u = 2 + 2 * t + k
            x = k
            wait_gather(x)
            drain_writes(u - 2, x)
            transpose_unit(x)
            issue_writes(u, x)
            u2 = lax.min(u + 2, _NU - 1)
            build_rowid(u2, x)
            issue_gather(x)
        return carry

    lax.fori_loop(0, (_NU - 2) // 2, body, 0)

    for k in range(2):
        wait_gather(k)
        drain_writes(_NU - 2 + k, k)


_lookup = pl.kernel(
    _lookup_body,
    out_type=jax.ShapeDtypeStruct((_SEQ, _D // 8, _NBB, 8, _BB), jnp.float32),
    mesh=plsc.VectorSubcoreMesh(core_axis_name="c", subcore_axis_name="s"),
    scratch_types=[
        pltpu.VMEM((_BPW * _SEQ,), jnp.int32),
        pltpu.VMEM((_BB,), jnp.int32),
        pltpu.VMEM((_BB,), jnp.int32),
        pltpu.VMEM((_BB, _D), jnp.float32),
        pltpu.VMEM((_BB, _D), jnp.float32),
        pltpu.VMEM((_BB, _D + 1), jnp.float32),
        pltpu.VMEM((_BB, _D + 1), jnp.float32),
        pltpu.VMEM((_D // 8, 8, _BB), jnp.float32),
        pltpu.VMEM((_D // 8, 8, _BB), jnp.float32),
        pltpu.SemaphoreType.DMA,
        pltpu.SemaphoreType.DMA,
        pltpu.SemaphoreType.DMA,
        pltpu.SemaphoreType.DMA,
    ],
    compiler_params=pltpu.CompilerParams(
        use_tc_tiling_on_sc=False, needs_layout_passes=False
    ),
)


@jax.jit
def kernel(idx, embedding):
    flat = idx.reshape(-1).astype(jnp.int32)
    out5 = _lookup(flat, embedding)
    out = out5.transpose(2, 4, 0, 1, 3).reshape(_BATCH, _SEQ, _D)
    return out
